# Initial kernel scaffold; baseline (speedup 1.0000x reference)
import jax
import jax.numpy as jnp
from jax.experimental import pallas as pl


def kernel(atomic_energy, atom_types, scales, shifts):
    raise NotImplementedError("write your pallas kernel here")


# speedup vs baseline: 3.9948x; 3.9948x over previous
import functools

import jax
import jax.numpy as jnp
from jax import lax
from jax.experimental import pallas as pl
from jax.experimental.pallas import tpu as pltpu
from jax.experimental.pallas import tpu_sc as plsc

_LANES = 16


@functools.lru_cache(maxsize=None)
def _build(n, num_types):
    info = plsc.get_sparse_core_info()
    nc, ns = info.num_cores, info.num_subcores
    nw = nc * ns
    chunk = -(-n // (8 * nw)) * 8
    assert chunk >= _LANES and n >= chunk and n % 8 == 0
    n_vecs = -(-chunk // _LANES)

    mesh = plsc.VectorSubcoreMesh(core_axis_name="c", subcore_axis_name="s")

    @functools.partial(
        pl.kernel,
        mesh=mesh,
        out_type=jax.ShapeDtypeStruct((n,), jnp.float32),
        compiler_params=pltpu.CompilerParams(needs_layout_passes=False),
        scratch_types=[
            pltpu.VMEM((chunk,), jnp.int32),
            pltpu.VMEM((chunk,), jnp.float32),
            pltpu.VMEM((chunk,), jnp.float32),
            pltpu.VMEM((num_types,), jnp.float32),
            pltpu.VMEM((num_types,), jnp.float32),
            pltpu.SemaphoreType.DMA,
        ],
    )
    def scale_shift(energy_hbm, types_hbm, scales_hbm, shifts_hbm, out_hbm,
                    idx_v, e_v, o_v, sc_v, sh_v, sem):
        wid = lax.axis_index("s") * nc + lax.axis_index("c")
        base = jnp.minimum(wid * chunk, n - chunk)
        cps = [
            pltpu.make_async_copy(scales_hbm, sc_v, sem),
            pltpu.make_async_copy(shifts_hbm, sh_v, sem),
            pltpu.make_async_copy(types_hbm.at[pl.ds(base, chunk)], idx_v, sem),
            pltpu.make_async_copy(energy_hbm.at[pl.ds(base, chunk)], e_v, sem),
        ]
        for cp in cps:
            cp.start()
        for cp in cps:
            cp.wait()

        def body(j, carry):
            off = jnp.minimum(j * _LANES, chunk - _LANES)
            idx16 = idx_v[pl.ds(off, _LANES)]
            e16 = e_v[pl.ds(off, _LANES)]
            sc16 = plsc.load_gather(sc_v, [idx16])
            sh16 = plsc.load_gather(sh_v, [idx16])
            o_v[pl.ds(off, _LANES)] = sh16 + sc16 * e16
            return carry

        lax.fori_loop(0, n_vecs, body, 0)
        pltpu.sync_copy(o_v, out_hbm.at[pl.ds(base, chunk)])

    return scale_shift


def kernel(atomic_energy, atom_types, scales, shifts):
    n = atom_types.shape[0]
    num_types = scales.shape[0]
    energy = atomic_energy.reshape(n).astype(jnp.float32)
    types = atom_types.astype(jnp.int32)
    out = _build(n, num_types)(energy, types, scales, shifts)
    return out.reshape(n, 1)


# speedup vs baseline: 4.0163x; 1.0054x over previous
import functools

import jax
import jax.numpy as jnp
from jax import lax
from jax.experimental import pallas as pl
from jax.experimental.pallas import tpu as pltpu
from jax.experimental.pallas import tpu_sc as plsc

_LANES = 16
_NB = 4


@functools.lru_cache(maxsize=None)
def _build(n, num_types):
    info = plsc.get_sparse_core_info()
    nc, ns = info.num_cores, info.num_subcores
    nw = nc * ns
    quant = _LANES * _NB
    chunk = -(-n // (quant * nw)) * quant
    sub = chunk // _NB
    n_vec = sub // _LANES
    assert n % 8 == 0 and n >= chunk

    mesh = plsc.VectorSubcoreMesh(core_axis_name="c", subcore_axis_name="s")

    @functools.partial(
        pl.kernel,
        mesh=mesh,
        out_type=jax.ShapeDtypeStruct((n,), jnp.float32),
        compiler_params=pltpu.CompilerParams(needs_layout_passes=False),
        scratch_types=[
            pltpu.VMEM((sub,), jnp.int32),
            pltpu.VMEM((sub,), jnp.int32),
            pltpu.VMEM((sub,), jnp.float32),
            pltpu.VMEM((sub,), jnp.float32),
            pltpu.VMEM((sub,), jnp.float32),
            pltpu.VMEM((sub,), jnp.float32),
            pltpu.VMEM((num_types,), jnp.float32),
            pltpu.VMEM((num_types,), jnp.float32),
            pltpu.SemaphoreType.DMA,
            pltpu.SemaphoreType.DMA,
            pltpu.SemaphoreType.DMA,
            pltpu.SemaphoreType.DMA,
            pltpu.SemaphoreType.DMA,
        ],
    )
    def scale_shift(energy_hbm, types_hbm, scales_hbm, shifts_hbm, out_hbm,
                    idx0, idx1, e0, e1, o0, o1, sc_v, sh_v,
                    sem_t, sem_i0, sem_i1, sem_o0, sem_o1):
        wid = lax.axis_index("s") * nc + lax.axis_index("c")
        base = jnp.minimum(wid * chunk, n - chunk)
        idx_v = (idx0, idx1)
        e_v = (e0, e1)
        o_v = (o0, o1)
        sem_i = (sem_i0, sem_i1)
        sem_o = (sem_o0, sem_o1)

        def in_copies(s):
            b = s % 2
            return (
                pltpu.make_async_copy(
                    types_hbm.at[pl.ds(base + s * sub, sub)], idx_v[b], sem_i[b]),
                pltpu.make_async_copy(
                    energy_hbm.at[pl.ds(base + s * sub, sub)], e_v[b], sem_i[b]),
            )

        tbl = (pltpu.make_async_copy(scales_hbm, sc_v, sem_t),
               pltpu.make_async_copy(shifts_hbm, sh_v, sem_t))
        for cp in tbl:
            cp.start()
        pend_in = {}
        for s in range(min(2, _NB)):
            pend_in[s] = in_copies(s)
            for cp in pend_in[s]:
                cp.start()
        for cp in tbl:
            cp.wait()

        pend_out = {}
        for s in range(_NB):
            b = s % 2
            for cp in pend_in.pop(s):
                cp.wait()
            if s >= 2:
                pend_out.pop(s - 2).wait()

            @plsc.parallel_loop(0, n_vec, unroll=7)
            def _vec(j, _b=b):
                off = j * _LANES
                idx16 = idx_v[_b][pl.ds(off, _LANES)]
                e16 = e_v[_b][pl.ds(off, _LANES)]
                sc16 = plsc.load_gather(sc_v, [idx16])
                sh16 = plsc.load_gather(sh_v, [idx16])
                o_v[_b][pl.ds(off, _LANES)] = sh16 + sc16 * e16

            co = pltpu.make_async_copy(
                o_v[b], out_hbm.at[pl.ds(base + s * sub, sub)], sem_o[b])
            co.start()
            pend_out[s] = co
            if s + 2 < _NB:
                pend_in[s + 2] = in_copies(s + 2)
                for cp in pend_in[s + 2]:
                    cp.start()
        for s in sorted(pend_out):
            pend_out[s].wait()

    return scale_shift


def kernel(atomic_energy, atom_types, scales, shifts):
    n = atom_types.shape[0]
    num_types = scales.shape[0]
    energy = atomic_energy.reshape(n).astype(jnp.float32)
    types = atom_types.astype(jnp.int32)
    out = _build(n, num_types)(energy, types, scales, shifts)
    return out.reshape(n, 1)


# speedup vs baseline: 4.0286x; 1.0031x over previous
import functools

import jax
import jax.numpy as jnp
from jax import lax
from jax.experimental import pallas as pl
from jax.experimental.pallas import tpu as pltpu
from jax.experimental.pallas import tpu_sc as plsc

_LANES = 16
_NB = 4


@functools.lru_cache(maxsize=None)
def _build(n, num_types):
    info = plsc.get_sparse_core_info()
    nc, ns = info.num_cores, info.num_subcores
    nw = nc * ns
    quant = _LANES * _NB
    chunk = -(-n // (quant * nw)) * quant
    sub = chunk // _NB
    n_vec = sub // _LANES
    assert n % 8 == 0 and n >= chunk

    mesh = plsc.VectorSubcoreMesh(core_axis_name="c", subcore_axis_name="s")

    @functools.partial(
        pl.kernel,
        mesh=mesh,
        out_type=jax.ShapeDtypeStruct((n,), jnp.float32),
        compiler_params=pltpu.CompilerParams(
            needs_layout_passes=False, skip_device_barrier=True),
        scratch_types=[
            pltpu.VMEM((sub,), jnp.int32),
            pltpu.VMEM((sub,), jnp.int32),
            pltpu.VMEM((sub,), jnp.float32),
            pltpu.VMEM((sub,), jnp.float32),
            pltpu.VMEM((sub,), jnp.float32),
            pltpu.VMEM((sub,), jnp.float32),
            pltpu.VMEM((num_types,), jnp.float32),
            pltpu.VMEM((num_types,), jnp.float32),
            pltpu.SemaphoreType.DMA,
            pltpu.SemaphoreType.DMA,
            pltpu.SemaphoreType.DMA,
            pltpu.SemaphoreType.DMA,
            pltpu.SemaphoreType.DMA,
        ],
    )
    def scale_shift(energy_hbm, types_hbm, scales_hbm, shifts_hbm, out_hbm,
                    idx0, idx1, e0, e1, o0, o1, sc_v, sh_v,
                    sem_t, sem_i0, sem_i1, sem_o0, sem_o1):
        wid = lax.axis_index("s") * nc + lax.axis_index("c")
        base = jnp.minimum(wid * chunk, n - chunk)
        idx_v = (idx0, idx1)
        e_v = (e0, e1)
        o_v = (o0, o1)
        sem_i = (sem_i0, sem_i1)
        sem_o = (sem_o0, sem_o1)

        def in_copies(s):
            b = s % 2
            return (
                pltpu.make_async_copy(
                    types_hbm.at[pl.ds(base + s * sub, sub)], idx_v[b], sem_i[b]),
                pltpu.make_async_copy(
                    energy_hbm.at[pl.ds(base + s * sub, sub)], e_v[b], sem_i[b]),
            )

        tbl = (pltpu.make_async_copy(scales_hbm, sc_v, sem_t),
               pltpu.make_async_copy(shifts_hbm, sh_v, sem_t))
        for cp in tbl:
            cp.start()
        pend_in = {}
        for s in range(min(2, _NB)):
            pend_in[s] = in_copies(s)
            for cp in pend_in[s]:
                cp.start()
        for cp in tbl:
            cp.wait()

        pend_out = {}
        for s in range(_NB):
            b = s % 2
            for cp in pend_in.pop(s):
                cp.wait()
            if s >= 2:
                pend_out.pop(s - 2).wait()

            @plsc.parallel_loop(0, n_vec, unroll=7)
            def _vec(j, _b=b):
                off = j * _LANES
                idx16 = idx_v[_b][pl.ds(off, _LANES)]
                e16 = e_v[_b][pl.ds(off, _LANES)]
                sc16 = plsc.load_gather(sc_v, [idx16])
                sh16 = plsc.load_gather(sh_v, [idx16])
                o_v[_b][pl.ds(off, _LANES)] = sh16 + sc16 * e16

            co = pltpu.make_async_copy(
                o_v[b], out_hbm.at[pl.ds(base + s * sub, sub)], sem_o[b])
            co.start()
            pend_out[s] = co
            if s + 2 < _NB:
                pend_in[s + 2] = in_copies(s + 2)
                for cp in pend_in[s + 2]:
                    cp.start()
        for s in sorted(pend_out):
            pend_out[s].wait()

    return scale_shift


def kernel(atomic_energy, atom_types, scales, shifts):
    n = atom_types.shape[0]
    num_types = scales.shape[0]
    energy = atomic_energy.reshape(n).astype(jnp.float32)
    types = atom_types.astype(jnp.int32)
    out = _build(n, num_types)(energy, types, scales, shifts)
    return out.reshape(n, 1)


# speedup vs baseline: 4.2214x; 1.0479x over previous
import functools

import jax
import jax.numpy as jnp
from jax import lax
from jax.experimental import pallas as pl
from jax.experimental.pallas import tpu as pltpu
from jax.experimental.pallas import tpu_sc as plsc

_LANES = 16
_NB = 4
_NUM_CORES = 1


@functools.lru_cache(maxsize=None)
def _build(n, num_types):
    info = plsc.get_sparse_core_info()
    nc, ns = _NUM_CORES, info.num_subcores
    nw = nc * ns
    quant = _LANES * _NB
    chunk = -(-n // (quant * nw)) * quant
    sub = chunk // _NB
    n_vec = sub // _LANES
    assert n % 8 == 0 and n >= chunk

    mesh = plsc.VectorSubcoreMesh(
        core_axis_name="c", subcore_axis_name="s", num_cores=nc)

    @functools.partial(
        pl.kernel,
        mesh=mesh,
        out_type=jax.ShapeDtypeStruct((n,), jnp.float32),
        compiler_params=pltpu.CompilerParams(needs_layout_passes=False),
        scratch_types=[
            pltpu.VMEM((sub,), jnp.int32),
            pltpu.VMEM((sub,), jnp.int32),
            pltpu.VMEM((sub,), jnp.float32),
            pltpu.VMEM((sub,), jnp.float32),
            pltpu.VMEM((sub,), jnp.float32),
            pltpu.VMEM((sub,), jnp.float32),
            pltpu.VMEM((num_types,), jnp.float32),
            pltpu.VMEM((num_types,), jnp.float32),
            pltpu.SemaphoreType.DMA,
            pltpu.SemaphoreType.DMA,
            pltpu.SemaphoreType.DMA,
            pltpu.SemaphoreType.DMA,
            pltpu.SemaphoreType.DMA,
        ],
    )
    def scale_shift(energy_hbm, types_hbm, scales_hbm, shifts_hbm, out_hbm,
                    idx0, idx1, e0, e1, o0, o1, sc_v, sh_v,
                    sem_t, sem_i0, sem_i1, sem_o0, sem_o1):
        wid = lax.axis_index("s") * nc + lax.axis_index("c")
        base = jnp.minimum(wid * chunk, n - chunk)
        idx_v = (idx0, idx1)
        e_v = (e0, e1)
        o_v = (o0, o1)
        sem_i = (sem_i0, sem_i1)
        sem_o = (sem_o0, sem_o1)

        def in_copies(s):
            b = s % 2
            return (
                pltpu.make_async_copy(
                    types_hbm.at[pl.ds(base + s * sub, sub)], idx_v[b], sem_i[b]),
                pltpu.make_async_copy(
                    energy_hbm.at[pl.ds(base + s * sub, sub)], e_v[b], sem_i[b]),
            )

        tbl = (pltpu.make_async_copy(scales_hbm, sc_v, sem_t),
               pltpu.make_async_copy(shifts_hbm, sh_v, sem_t))
        for cp in tbl:
            cp.start()
        pend_in = {}
        for s in range(min(2, _NB)):
            pend_in[s] = in_copies(s)
            for cp in pend_in[s]:
                cp.start()
        for cp in tbl:
            cp.wait()

        pend_out = {}
        for s in range(_NB):
            b = s % 2
            for cp in pend_in.pop(s):
                cp.wait()
            if s >= 2:
                pend_out.pop(s - 2).wait()

            @plsc.parallel_loop(0, n_vec, unroll=7)
            def _vec(j, _b=b):
                off = j * _LANES
                idx16 = idx_v[_b][pl.ds(off, _LANES)]
                e16 = e_v[_b][pl.ds(off, _LANES)]
                sc16 = plsc.load_gather(sc_v, [idx16])
                sh16 = plsc.load_gather(sh_v, [idx16])
                o_v[_b][pl.ds(off, _LANES)] = sh16 + sc16 * e16

            co = pltpu.make_async_copy(
                o_v[b], out_hbm.at[pl.ds(base + s * sub, sub)], sem_o[b])
            co.start()
            pend_out[s] = co
            if s + 2 < _NB:
                pend_in[s + 2] = in_copies(s + 2)
                for cp in pend_in[s + 2]:
                    cp.start()
        for s in sorted(pend_out):
            pend_out[s].wait()

    return scale_shift


def kernel(atomic_energy, atom_types, scales, shifts):
    n = atom_types.shape[0]
    num_types = scales.shape[0]
    energy = atomic_energy.reshape(n).astype(jnp.float32)
    types = atom_types.astype(jnp.int32)
    out = _build(n, num_types)(energy, types, scales, shifts)
    return out.reshape(n, 1)


# speedup vs baseline: 4.3669x; 1.0345x over previous
import functools

import jax
import jax.numpy as jnp
from jax import lax
from jax.experimental import pallas as pl
from jax.experimental.pallas import tpu as pltpu
from jax.experimental.pallas import tpu_sc as plsc

_LANES = 16
_NB = 2
_NUM_CORES = 1


@functools.lru_cache(maxsize=None)
def _build(n, num_types):
    info = plsc.get_sparse_core_info()
    nc, ns = _NUM_CORES, info.num_subcores
    nw = nc * ns
    quant = _LANES * _NB
    chunk = -(-n // (quant * nw)) * quant
    sub = chunk // _NB
    n_vec = sub // _LANES
    assert n % 8 == 0 and n >= chunk

    mesh = plsc.VectorSubcoreMesh(
        core_axis_name="c", subcore_axis_name="s", num_cores=nc)

    @functools.partial(
        pl.kernel,
        mesh=mesh,
        out_type=jax.ShapeDtypeStruct((n,), jnp.float32),
        compiler_params=pltpu.CompilerParams(needs_layout_passes=False),
        scratch_types=[
            pltpu.VMEM((sub,), jnp.int32),
            pltpu.VMEM((sub,), jnp.int32),
            pltpu.VMEM((sub,), jnp.float32),
            pltpu.VMEM((sub,), jnp.float32),
            pltpu.VMEM((sub,), jnp.float32),
            pltpu.VMEM((sub,), jnp.float32),
            pltpu.VMEM((num_types,), jnp.float32),
            pltpu.VMEM((num_types,), jnp.float32),
            pltpu.SemaphoreType.DMA,
            pltpu.SemaphoreType.DMA,
            pltpu.SemaphoreType.DMA,
            pltpu.SemaphoreType.DMA,
            pltpu.SemaphoreType.DMA,
        ],
    )
    def scale_shift(energy_hbm, types_hbm, scales_hbm, shifts_hbm, out_hbm,
                    idx0, idx1, e0, e1, o0, o1, sc_v, sh_v,
                    sem_t, sem_i0, sem_i1, sem_o0, sem_o1):
        wid = lax.axis_index("s") * nc + lax.axis_index("c")
        base = jnp.minimum(wid * chunk, n - chunk)
        idx_v = (idx0, idx1)
        e_v = (e0, e1)
        o_v = (o0, o1)
        sem_i = (sem_i0, sem_i1)
        sem_o = (sem_o0, sem_o1)

        def in_copies(s):
            b = s % 2
            return (
                pltpu.make_async_copy(
                    types_hbm.at[pl.ds(base + s * sub, sub)], idx_v[b], sem_i[b]),
                pltpu.make_async_copy(
                    energy_hbm.at[pl.ds(base + s * sub, sub)], e_v[b], sem_i[b]),
            )

        tbl = (pltpu.make_async_copy(scales_hbm, sc_v, sem_t),
               pltpu.make_async_copy(shifts_hbm, sh_v, sem_t))
        for cp in tbl:
            cp.start()
        pend_in = {}
        for s in range(min(2, _NB)):
            pend_in[s] = in_copies(s)
            for cp in pend_in[s]:
                cp.start()
        for cp in tbl:
            cp.wait()

        pend_out = {}
        for s in range(_NB):
            b = s % 2
            for cp in pend_in.pop(s):
                cp.wait()
            if s >= 2:
                pend_out.pop(s - 2).wait()

            @plsc.parallel_loop(0, n_vec, unroll=7)
            def _vec(j, _b=b):
                off = j * _LANES
                idx16 = idx_v[_b][pl.ds(off, _LANES)]
                e16 = e_v[_b][pl.ds(off, _LANES)]
                sc16 = plsc.load_gather(sc_v, [idx16])
                sh16 = plsc.load_gather(sh_v, [idx16])
                o_v[_b][pl.ds(off, _LANES)] = sh16 + sc16 * e16

            co = pltpu.make_async_copy(
                o_v[b], out_hbm.at[pl.ds(base + s * sub, sub)], sem_o[b])
            co.start()
            pend_out[s] = co
            if s + 2 < _NB:
                pend_in[s + 2] = in_copies(s + 2)
                for cp in pend_in[s + 2]:
                    cp.start()
        for s in sorted(pend_out):
            pend_out[s].wait()

    return scale_shift


def kernel(atomic_energy, atom_types, scales, shifts):
    n = atom_types.shape[0]
    num_types = scales.shape[0]
    energy = atomic_energy.reshape(n).astype(jnp.float32)
    types = atom_types.astype(jnp.int32)
    out = _build(n, num_types)(energy, types, scales, shifts)
    return out.reshape(n, 1)
